# Initial kernel scaffold; baseline (speedup 1.0000x reference)
import jax
import jax.numpy as jnp
from jax.experimental import pallas as pl


def kernel(position_ids, table):
    raise NotImplementedError("write your pallas kernel here")


# speedup vs baseline: 2.5325x; 2.5325x over previous
import functools

import jax
import jax.numpy as jnp
from jax import lax
from jax.experimental import pallas as pl
from jax.experimental.pallas import tpu as pltpu
from jax.experimental.pallas import tpu_sc as plsc

HIDDEN = 768
NUM_CORES = 2
NUM_SUBCORES = 16
NW = NUM_CORES * NUM_SUBCORES

CHUNK = 64


def _sc_gather(table, idx_flat, b_total):
    b_per_w = b_total // NW
    n_chunks = b_per_w // CHUNK
    mesh = plsc.VectorSubcoreMesh(core_axis_name="c", subcore_axis_name="s")

    @functools.partial(
        pl.kernel,
        mesh=mesh,
        out_type=jax.ShapeDtypeStruct((b_total, HIDDEN), jnp.float32),
        scratch_types=[
            pltpu.VMEM((b_per_w,), jnp.int32),
            pltpu.VMEM((CHUNK, HIDDEN), jnp.float32),
            pltpu.VMEM((CHUNK, HIDDEN), jnp.float32),
            pltpu.SemaphoreType.DMA,
            pltpu.SemaphoreType.DMA,
        ],
    )
    def k(table_hbm, idx_hbm, out_hbm, idx_v, buf0, buf1, sem0, sem1):
        wid = lax.axis_index("s") * NUM_CORES + lax.axis_index("c")
        base = wid * b_per_w
        pltpu.sync_copy(idx_hbm.at[pl.ds(base, b_per_w)], idx_v)

        pltpu.async_copy(table_hbm.at[idx_v.at[pl.ds(0, CHUNK)]], buf0, sem0)
        pltpu.async_copy(
            table_hbm.at[idx_v.at[pl.ds(CHUNK, CHUNK)]], buf1, sem1
        )

        def step(c, buf, sem):
            pltpu.make_async_copy(
                table_hbm.at[idx_v.at[pl.ds(c * CHUNK, CHUNK)]], buf, sem
            ).wait()
            pltpu.sync_copy(buf, out_hbm.at[pl.ds(base + c * CHUNK, CHUNK)])

            @pl.when(c + 2 < n_chunks)
            def _():
                pltpu.async_copy(
                    table_hbm.at[idx_v.at[pl.ds((c + 2) * CHUNK, CHUNK)]],
                    buf,
                    sem,
                )

        def body(i, carry):
            step(2 * i, buf0, sem0)
            step(2 * i + 1, buf1, sem1)
            return carry

        lax.fori_loop(0, n_chunks // 2, body, 0)

    return k(table, idx_flat)


def kernel(position_ids, table):
    batch, seq = position_ids.shape
    b_total = batch * seq
    idx_flat = position_ids.reshape(b_total).astype(jnp.int32)
    out = _sc_gather(table, idx_flat, b_total)
    return out.reshape(batch, seq, HIDDEN)
